# baseline (speedup 1.0000x reference)
import functools

import jax
import jax.numpy as jnp
from jax import lax
from jax.experimental import pallas as pl
from jax.experimental.pallas import tpu as pltpu
from jax.experimental.pallas import tpu_sc as plsc

B, S, D = 4, 2048, 1024
_NC = 2
_NW = 32
_SPW = S // _NW
_TROWS = 16
_NT = _SPW // _TROWS


@functools.partial(
    pl.kernel,
    mesh=plsc.VectorSubcoreMesh(core_axis_name="c", subcore_axis_name="s"),
    out_type=jax.ShapeDtypeStruct((B, S, D), jnp.float32),
    scratch_types=[
        pltpu.VMEM((_SPW, D), jnp.float32),
        pltpu.VMEM((_TROWS, D), jnp.float32),
    ],
)
def _sc_add(x_hbm, pe_hbm, out_hbm, pe_v, xt_v):
    wid = lax.axis_index("s") * _NC + lax.axis_index("c")
    base = wid * _SPW
    pltpu.sync_copy(pe_hbm.at[pl.ds(base, _SPW)], pe_v)
    vregs_per_row = D // 16
    for b in range(B):
        for t in range(_NT):
            pltpu.sync_copy(x_hbm.at[b, pl.ds(base + t * _TROWS, _TROWS)], xt_v)

            def add_body(i, _, t=t):
                r = i // vregs_per_row
                c = (i % vregs_per_row) * 16
                xt_v[r, pl.ds(c, 16)] = (
                    xt_v[r, pl.ds(c, 16)] + pe_v[t * _TROWS + r, pl.ds(c, 16)]
                )
                return 0

            lax.fori_loop(0, _TROWS * vregs_per_row, add_body, 0)
            pltpu.sync_copy(xt_v, out_hbm.at[b, pl.ds(base + t * _TROWS, _TROWS)])


_BS = 256


def _tc_add_body(x_ref, pe_ref, o_ref):
    o_ref[...] = x_ref[...] + pe_ref[...]


def _tc_add(x, pe):
    b, s, d = x.shape
    return pl.pallas_call(
        _tc_add_body,
        grid=(s // _BS, b),
        in_specs=[
            pl.BlockSpec((1, _BS, d), lambda i, j: (j, i, 0)),
            pl.BlockSpec((_BS, d), lambda i, j: (i, 0)),
        ],
        out_specs=pl.BlockSpec((1, _BS, d), lambda i, j: (j, i, 0)),
        out_shape=jax.ShapeDtypeStruct((b, s, d), x.dtype),
    )(x, pe)


def kernel(x, pe):
    return _sc_add(x, pe[:S])


# speedup vs baseline: 2.5137x; 2.5137x over previous
import functools

import jax
import jax.numpy as jnp
from jax import lax
from jax.experimental import pallas as pl
from jax.experimental.pallas import tpu as pltpu
from jax.experimental.pallas import tpu_sc as plsc

B, S, D = 4, 2048, 1024
_NC = 2
_NW = 32
_SPW = S // _NW
_TROWS = 16
_NT = _SPW // _TROWS


@functools.partial(
    pl.kernel,
    mesh=plsc.VectorSubcoreMesh(core_axis_name="c", subcore_axis_name="s"),
    out_type=jax.ShapeDtypeStruct((B, S, D), jnp.float32),
    scratch_types=[
        pltpu.VMEM((_SPW, D), jnp.float32),
        pltpu.VMEM((_TROWS, D), jnp.float32),
        pltpu.VMEM((_TROWS, D), jnp.float32),
        pltpu.VMEM((_TROWS, D), jnp.float32),
        pltpu.SemaphoreType.DMA,
        pltpu.SemaphoreType.DMA,
        pltpu.SemaphoreType.DMA,
        pltpu.SemaphoreType.DMA,
        pltpu.SemaphoreType.DMA,
        pltpu.SemaphoreType.DMA,
    ],
)
def _sc_add(x_hbm, pe_hbm, out_hbm, pe_v, xa, xb, xc, sia, sib, sic, soa, sob, soc):
    wid = lax.axis_index("s") * _NC + lax.axis_index("c")
    base = wid * _SPW
    tiles = [(b, t) for b in range(B) for t in range(_NT)]
    bufs = [(xa, sia, soa), (xb, sib, sob), (xc, sic, soc)]
    n = len(tiles)
    in_dma = [None, None, None]
    out_dma = [None, None, None]
    b0, t0 = tiles[0]
    in_dma[0] = pltpu.async_copy(
        x_hbm.at[b0, pl.ds(base + t0 * _TROWS, _TROWS)], xa, sia
    )
    pltpu.sync_copy(pe_hbm.at[pl.ds(base, _SPW)], pe_v)
    for k, (b, t) in enumerate(tiles):
        cur = k % 3
        buf, _, sout = bufs[cur]
        in_dma[cur].wait()
        if k + 1 < n:
            nb, nt = tiles[k + 1]
            nxt = (k + 1) % 3
            nbuf, nsin, _ = bufs[nxt]
            if out_dma[nxt] is not None:
                out_dma[nxt].wait()
            in_dma[nxt] = pltpu.async_copy(
                x_hbm.at[nb, pl.ds(base + nt * _TROWS, _TROWS)], nbuf, nsin
            )

        @plsc.parallel_loop(0, _TROWS)
        def row_body(r, buf=buf, t=t):
            @plsc.parallel_loop(0, D, step=16, unroll=8)
            def col_body(c):
                buf[r, pl.ds(c, 16)] = (
                    buf[r, pl.ds(c, 16)] + pe_v[t * _TROWS + r, pl.ds(c, 16)]
                )

        out_dma[cur] = pltpu.async_copy(
            buf, out_hbm.at[b, pl.ds(base + t * _TROWS, _TROWS)], sout
        )
    for d in out_dma:
        if d is not None:
            d.wait()


_BS = 256


def _tc_add_body(x_ref, pe_ref, o_ref):
    o_ref[...] = x_ref[...] + pe_ref[...]


def _tc_add(x, pe):
    b, s, d = x.shape
    return pl.pallas_call(
        _tc_add_body,
        grid=(s // _BS, b),
        in_specs=[
            pl.BlockSpec((1, _BS, d), lambda i, j: (j, i, 0)),
            pl.BlockSpec((_BS, d), lambda i, j: (i, 0)),
        ],
        out_specs=pl.BlockSpec((1, _BS, d), lambda i, j: (j, i, 0)),
        out_shape=jax.ShapeDtypeStruct((b, s, d), x.dtype),
    )(x, pe)


def kernel(x, pe):
    return _sc_add(x, pe[:S])
